# baseline (speedup 1.0000x reference)
import functools

import jax
import jax.numpy as jnp
from jax import lax
from jax.experimental import pallas as pl
from jax.experimental.pallas import tpu as pltpu
from jax.experimental.pallas import tpu_sc as plsc

N = 10000
E = 320000
D = 128

NC = 2
NS = 16
NW = NC * NS

NPAD = 10112
RPT = NPAD // NS
CHUNK = 128
NCHUNK = 84
EPT = NCHUNK * CHUNK
EPAD = NW * EPT
NBUF = 3

_mesh = plsc.VectorSubcoreMesh(core_axis_name="c", subcore_axis_name="s")


@functools.partial(
    pl.kernel,
    out_type=jax.ShapeDtypeStruct((NC, NPAD, D), jnp.float32),
    mesh=_mesh,
    scratch_types=[
        [pltpu.VMEM((CHUNK,), jnp.int32) for _ in range(2)],
        pltpu.VMEM((CHUNK, D), jnp.float32),
        pltpu.VMEM_SHARED((NPAD, D), jnp.float32),
        [pltpu.SemaphoreType.DMA for _ in range(2)],
    ],
)
def _deg_kernel(dst_hbm, ones_hbm, zeros_hbm, out_hbm, idx_v, ones_v, deg_sh,
                sem_i):
    cid = lax.axis_index("c")
    sid = lax.axis_index("s")
    t = cid * NS + sid
    pltpu.sync_copy(ones_hbm, ones_v)
    pltpu.sync_copy(zeros_hbm, deg_sh.at[pl.ds(sid * RPT, RPT)])
    plsc.subcore_barrier()

    pltpu.async_copy(dst_hbm.at[t, 0], idx_v[0], sem_i[0])
    pltpu.async_copy(dst_hbm.at[t, 1], idx_v[1], sem_i[1])

    def step(i, carry):
        for b in range(2):
            c = 2 * i + b
            pltpu.make_async_copy(dst_hbm.at[t, 0], idx_v[b], sem_i[b]).wait()
            pltpu.sync_copy(ones_v, deg_sh.at[idx_v[b]], add=True)

            @pl.when(c + 2 < NCHUNK)
            def _():
                pltpu.async_copy(dst_hbm.at[t, c + 2], idx_v[b], sem_i[b])
        return carry

    lax.fori_loop(0, NCHUNK // 2, step, 0)
    plsc.subcore_barrier()
    pltpu.sync_copy(deg_sh.at[pl.ds(sid * RPT, RPT)],
                    out_hbm.at[cid, pl.ds(sid * RPT, RPT)])


@functools.partial(
    pl.kernel,
    out_type=jax.ShapeDtypeStruct((NC, NPAD, D), jnp.float32),
    mesh=_mesh,
    scratch_types=[
        [pltpu.VMEM((CHUNK,), jnp.int32) for _ in range(NBUF)],
        [pltpu.VMEM((CHUNK,), jnp.int32) for _ in range(NBUF)],
        [pltpu.VMEM((CHUNK, D), jnp.float32) for _ in range(NBUF)],
        pltpu.VMEM_SHARED((NPAD, D), jnp.float32),
        [pltpu.SemaphoreType.DMA for _ in range(NBUF)],
        [pltpu.SemaphoreType.DMA for _ in range(NBUF)],
    ],
)
def _agg_kernel(g_hbm, src_hbm, dst_hbm, zeros_hbm, out_hbm,
                src_v, dst_v, rows_v, s_sh, sem_i, sem_g):
    cid = lax.axis_index("c")
    sid = lax.axis_index("s")
    t = cid * NS + sid
    pltpu.sync_copy(zeros_hbm, s_sh.at[pl.ds(sid * RPT, RPT)])

    def stage(c, b):
        pltpu.async_copy(src_hbm.at[t, c], src_v[b], sem_i[b])
        pltpu.async_copy(dst_hbm.at[t, c], dst_v[b], sem_i[b])

    def wait_stage(b):
        pltpu.make_async_copy(src_hbm.at[t, 0], src_v[b], sem_i[b]).wait()
        pltpu.make_async_copy(dst_hbm.at[t, 0], dst_v[b], sem_i[b]).wait()

    def gather(c, b):
        pltpu.async_copy(g_hbm.at[src_v[b]], rows_v[b], sem_g[b])

    def wait_gather(b):
        pltpu.make_async_copy(g_hbm.at[src_v[b]], rows_v[b], sem_g[b]).wait()

    for b in range(NBUF):
        stage(b, b)
    plsc.subcore_barrier()
    for b in range(NBUF - 1):
        wait_stage(b)
        gather(b, b)

    def step(i, carry):
        for b in range(NBUF):
            c = NBUF * i + b
            pb = (b - 1) % NBUF

            @pl.when(c + NBUF - 1 < NCHUNK)
            def _():
                wait_stage(pb)
                gather(c + NBUF - 1, pb)

            wait_gather(b)
            pltpu.sync_copy(rows_v[b], s_sh.at[dst_v[b]], add=True)

            @pl.when(c + NBUF < NCHUNK)
            def _():
                stage(c + NBUF, b)
        return carry

    lax.fori_loop(0, NCHUNK // NBUF, step, 0)
    plsc.subcore_barrier()
    pltpu.sync_copy(s_sh.at[pl.ds(sid * RPT, RPT)],
                    out_hbm.at[cid, pl.ds(sid * RPT, RPT)])


_R = 1264


def _dinv_of(dp):
    deg = 1.0 + dp[0, :, :1] + dp[1, :, :1]
    return lax.rsqrt(deg)


def _pre_body(x_ref, w_ref, dp_ref, h_ref, g_ref):
    dinv = _dinv_of(dp_ref[...])
    h = jnp.dot(x_ref[...], w_ref[...], preferred_element_type=jnp.float32)
    h_ref[...] = h
    g_ref[...] = h * dinv


def _mid_body(s_ref, h_ref, dp_ref, b_ref, w_ref, h2_ref, g2_ref):
    i = pl.program_id(0)
    dinv = _dinv_of(dp_ref[...])
    s = s_ref[0] + s_ref[1]
    pre = dinv * s + dinv * dinv * h_ref[...] + b_ref[...]
    rows = i * _R + lax.broadcasted_iota(jnp.int32, (_R, 1), 0)
    a = jnp.where(rows < N, jnp.maximum(pre, 0.0), 0.0)
    h2 = jnp.dot(a, w_ref[...], preferred_element_type=jnp.float32)
    h2_ref[...] = h2
    g2_ref[...] = h2 * dinv


def _post_body(s_ref, h_ref, dp_ref, b_ref, out_ref):
    dinv = _dinv_of(dp_ref[...])
    s = s_ref[0] + s_ref[1]
    pre = dinv * s + dinv * dinv * h_ref[...] + b_ref[...]
    out_ref[...] = jnp.maximum(pre, 0.0)


_spec_rows = pl.BlockSpec((_R, D), lambda i: (i, 0))
_spec_w = pl.BlockSpec((D, D), lambda i: (0, 0))
_spec_dp = pl.BlockSpec((2, _R, D), lambda i: (0, i, 0))
_spec_s = pl.BlockSpec((2, _R, D), lambda i: (0, i, 0))
_spec_b = pl.BlockSpec((1, D), lambda i: (0, 0))
_grid = (NPAD // _R,)
_f32 = jnp.float32


def _tc_pre(x, w1, dp):
    return pl.pallas_call(
        _pre_body, grid=_grid,
        in_specs=[_spec_rows, _spec_w, _spec_dp],
        out_specs=[_spec_rows, _spec_rows],
        out_shape=[jax.ShapeDtypeStruct((NPAD, D), _f32)] * 2,
    )(x, w1, dp)


def _tc_mid(s, h, dp, b1, w2):
    return pl.pallas_call(
        _mid_body, grid=_grid,
        in_specs=[_spec_s, _spec_rows, _spec_dp, _spec_b, _spec_w],
        out_specs=[_spec_rows, _spec_rows],
        out_shape=[jax.ShapeDtypeStruct((NPAD, D), _f32)] * 2,
    )(s, h, dp, b1, w2)


def _tc_post(s, h, dp, b2):
    return pl.pallas_call(
        _post_body, grid=_grid,
        in_specs=[_spec_s, _spec_rows, _spec_dp, _spec_b],
        out_specs=_spec_rows,
        out_shape=jax.ShapeDtypeStruct((NPAD, D), _f32),
    )(s, h, dp, b2)


def kernel(x, edge_index, W1, b1, W2, b2):
    src = edge_index[0].astype(jnp.int32)
    dst = edge_index[1].astype(jnp.int32)
    pad = jnp.full((EPAD - E,), N, dtype=jnp.int32)
    src_r = jnp.concatenate([src, pad]).reshape(NW, NCHUNK, CHUNK)
    dst_r = jnp.concatenate([dst, pad]).reshape(NW, NCHUNK, CHUNK)

    x_pad = jnp.pad(x, ((0, NPAD - N), (0, 0)))
    ones128 = jnp.ones((CHUNK, D), jnp.float32)
    zeros128 = jnp.zeros((RPT, D), jnp.float32)
    b1r = b1.reshape(1, D)
    b2r = b2.reshape(1, D)

    dp = _deg_kernel(dst_r, ones128, zeros128)
    h1, g1 = _tc_pre(x_pad, W1, dp)
    s1 = _agg_kernel(g1, src_r, dst_r, zeros128)
    h2, g2 = _tc_mid(s1, h1, dp, b1r, W2)
    s2 = _agg_kernel(g2, src_r, dst_r, zeros128)
    out = _tc_post(s2, h2, dp, b2r)
    return (out[:N], edge_index)


# speedup vs baseline: 3.0545x; 3.0545x over previous
import functools

import jax
import jax.numpy as jnp
from jax import lax
from jax.experimental import pallas as pl
from jax.experimental.pallas import tpu as pltpu
from jax.experimental.pallas import tpu_sc as plsc

N = 10000
E = 320000
D = 128

NC = 2
NS = 16
NW = NC * NS

NPAD = 10240
RPT = NPAD // NS
CHUNK = 128
CT = 2528
EPAD = CT * CHUNK

AC0 = 118
AC1 = 40

DC = CT // NW

_mesh = plsc.VectorSubcoreMesh(core_axis_name="c", subcore_axis_name="s")


@functools.partial(
    pl.kernel,
    out_type=jax.ShapeDtypeStruct((NC, NPAD, D), jnp.float32),
    mesh=_mesh,
    scratch_types=[
        [pltpu.VMEM((CHUNK,), jnp.int32) for _ in range(2)],
        pltpu.VMEM((CHUNK, D), jnp.float32),
        pltpu.VMEM_SHARED((NPAD, D), jnp.float32),
        [pltpu.SemaphoreType.DMA for _ in range(2)],
    ],
)
def _deg_kernel(dst_hbm, ones_hbm, zeros_hbm, out_hbm, idx_v, ones_v, deg_sh,
                sem_i):
    cid = lax.axis_index("c")
    sid = lax.axis_index("s")
    base = (cid * NS + sid) * DC
    pltpu.sync_copy(ones_hbm, ones_v)
    pltpu.sync_copy(zeros_hbm, deg_sh.at[pl.ds(sid * RPT, RPT)])
    plsc.subcore_barrier()

    pltpu.async_copy(dst_hbm.at[base + 0], idx_v[0], sem_i[0])
    pltpu.async_copy(dst_hbm.at[base + 1], idx_v[1], sem_i[1])

    def step(i, carry):
        for b in range(2):
            c = 2 * i + b
            pltpu.make_async_copy(dst_hbm.at[base], idx_v[b], sem_i[b]).wait()
            pltpu.sync_copy(ones_v, deg_sh.at[idx_v[b]], add=True)

            @pl.when(c + 2 < DC)
            def _():
                pltpu.async_copy(dst_hbm.at[base + c + 2], idx_v[b], sem_i[b])
        return carry

    lax.fori_loop(0, DC // 2, step, 0)
    pltpu.make_async_copy(dst_hbm.at[base], idx_v[0], sem_i[0]).wait()
    pltpu.sync_copy(ones_v, deg_sh.at[idx_v[0]], add=True)
    plsc.subcore_barrier()
    pltpu.sync_copy(deg_sh.at[pl.ds(sid * RPT, RPT)],
                    out_hbm.at[cid, pl.ds(sid * RPT, RPT)])


@functools.partial(
    pl.kernel,
    out_type=jax.ShapeDtypeStruct((NC, NPAD, D), jnp.float32),
    mesh=_mesh,
    scratch_types=[
        [pltpu.VMEM((CHUNK,), jnp.int32) for _ in range(2)],
        [pltpu.VMEM((CHUNK,), jnp.int32) for _ in range(2)],
        [pltpu.VMEM((CHUNK, D), jnp.float32) for _ in range(2)],
        pltpu.VMEM_SHARED((NPAD, D), jnp.float32),
        [pltpu.SemaphoreType.DMA for _ in range(2)],
        [pltpu.SemaphoreType.DMA for _ in range(2)],
    ],
)
def _agg_kernel(g_hbm, src_hbm, dst_hbm, zeros_hbm, out_hbm,
                src_v, dst_v, rows_v, s_sh, sem_i, sem_g):
    cid = lax.axis_index("c")
    sid = lax.axis_index("s")
    nc = lax.select(cid == 0, AC0, AC1)
    base = lax.select(cid == 0, NS * AC1 + sid * AC0, sid * AC1)
    pltpu.sync_copy(zeros_hbm, s_sh.at[pl.ds(sid * RPT, RPT)])

    def stage(c, b):
        pltpu.async_copy(src_hbm.at[base + c], src_v[b], sem_i[b])
        pltpu.async_copy(dst_hbm.at[base + c], dst_v[b], sem_i[b])

    def wait_stage(b):
        pltpu.make_async_copy(src_hbm.at[base], src_v[b], sem_i[b]).wait()
        pltpu.make_async_copy(dst_hbm.at[base], dst_v[b], sem_i[b]).wait()

    def gather(c, b):
        pltpu.async_copy(g_hbm.at[src_v[b]], rows_v[b], sem_g[b])

    def wait_gather(b):
        pltpu.make_async_copy(g_hbm.at[src_v[b]], rows_v[b], sem_g[b]).wait()

    stage(0, 0)
    stage(1, 1)
    plsc.subcore_barrier()
    wait_stage(0)
    gather(0, 0)

    def step(i, carry):
        for b in range(2):
            c = 2 * i + b
            nb = 1 - b

            @pl.when(c < nc)
            def _():
                @pl.when(c + 1 < nc)
                def _():
                    wait_stage(nb)
                    gather(c + 1, nb)

                wait_gather(b)
                pltpu.sync_copy(rows_v[b], s_sh.at[dst_v[b]], add=True)

                @pl.when(c + 2 < nc)
                def _():
                    stage(c + 2, b)
        return carry

    lax.fori_loop(0, AC0 // 2, step, 0)
    plsc.subcore_barrier()
    pltpu.sync_copy(s_sh.at[pl.ds(sid * RPT, RPT)],
                    out_hbm.at[cid, pl.ds(sid * RPT, RPT)])


_R = 1280


def _dinv_of(dp):
    deg = 1.0 + dp[0, :, :1] + dp[1, :, :1]
    return lax.rsqrt(deg)


def _pre_body(x_ref, w_ref, dp_ref, h_ref, g_ref):
    dinv = _dinv_of(dp_ref[...])
    h = jnp.dot(x_ref[...], w_ref[...], preferred_element_type=jnp.float32)
    h_ref[...] = h
    g_ref[...] = h * dinv


def _mid_body(s_ref, h_ref, dp_ref, b_ref, w_ref, h2_ref, g2_ref):
    i = pl.program_id(0)
    dinv = _dinv_of(dp_ref[...])
    s = s_ref[0] + s_ref[1]
    pre = dinv * s + dinv * dinv * h_ref[...] + b_ref[...]
    rows = i * _R + lax.broadcasted_iota(jnp.int32, (_R, 1), 0)
    a = jnp.where(rows < N, jnp.maximum(pre, 0.0), 0.0)
    h2 = jnp.dot(a, w_ref[...], preferred_element_type=jnp.float32)
    h2_ref[...] = h2
    g2_ref[...] = h2 * dinv


def _post_body(s_ref, h_ref, dp_ref, b_ref, out_ref):
    dinv = _dinv_of(dp_ref[...])
    s = s_ref[0] + s_ref[1]
    pre = dinv * s + dinv * dinv * h_ref[...] + b_ref[...]
    out_ref[...] = jnp.maximum(pre, 0.0)


_spec_rows = pl.BlockSpec((_R, D), lambda i: (i, 0))
_spec_w = pl.BlockSpec((D, D), lambda i: (0, 0))
_spec_dp = pl.BlockSpec((2, _R, D), lambda i: (0, i, 0))
_spec_s = pl.BlockSpec((2, _R, D), lambda i: (0, i, 0))
_spec_b = pl.BlockSpec((1, D), lambda i: (0, 0))
_grid = (NPAD // _R,)
_f32 = jnp.float32


def _tc_pre(x, w1, dp):
    return pl.pallas_call(
        _pre_body, grid=_grid,
        in_specs=[_spec_rows, _spec_w, _spec_dp],
        out_specs=[_spec_rows, _spec_rows],
        out_shape=[jax.ShapeDtypeStruct((NPAD, D), _f32)] * 2,
    )(x, w1, dp)


def _tc_mid(s, h, dp, b1, w2):
    return pl.pallas_call(
        _mid_body, grid=_grid,
        in_specs=[_spec_s, _spec_rows, _spec_dp, _spec_b, _spec_w],
        out_specs=[_spec_rows, _spec_rows],
        out_shape=[jax.ShapeDtypeStruct((NPAD, D), _f32)] * 2,
    )(s, h, dp, b1, w2)


def _tc_post(s, h, dp, b2):
    return pl.pallas_call(
        _post_body, grid=_grid,
        in_specs=[_spec_s, _spec_rows, _spec_dp, _spec_b],
        out_specs=_spec_rows,
        out_shape=jax.ShapeDtypeStruct((NPAD, D), _f32),
    )(s, h, dp, b2)


def kernel(x, edge_index, W1, b1, W2, b2):
    src = edge_index[0].astype(jnp.int32)
    dst = edge_index[1].astype(jnp.int32)
    pad = jnp.full((EPAD - E,), N, dtype=jnp.int32)
    src_r = jnp.concatenate([src, pad]).reshape(CT, CHUNK)
    dst_r = jnp.concatenate([dst, pad]).reshape(CT, CHUNK)

    x_pad = jnp.pad(x, ((0, NPAD - N), (0, 0)))
    ones128 = jnp.ones((CHUNK, D), jnp.float32)
    zeros128 = jnp.zeros((RPT, D), jnp.float32)
    b1r = b1.reshape(1, D)
    b2r = b2.reshape(1, D)

    dp = _deg_kernel(dst_r, ones128, zeros128)
    h1, g1 = _tc_pre(x_pad, W1, dp)
    s1 = _agg_kernel(g1, src_r, dst_r, zeros128)
    h2, g2 = _tc_mid(s1, h1, dp, b1r, W2)
    s2 = _agg_kernel(g2, src_r, dst_r, zeros128)
    out = _tc_post(s2, h2, dp, b2r)
    return (out[:N], edge_index)


# speedup vs baseline: 3.1818x; 1.0416x over previous
import functools

import jax
import jax.numpy as jnp
from jax import lax
from jax.experimental import pallas as pl
from jax.experimental.pallas import tpu as pltpu
from jax.experimental.pallas import tpu_sc as plsc

N = 10000
E = 320000
D = 128

NC = 2
NS = 16
NW = NC * NS

NPAD = 10240
RPT = NPAD // NS
CHUNK = 128
CT = 2528
EPAD = CT * CHUNK

AC0 = 40
AC1 = 118
ACMAX = 118

DC = CT // NW

_mesh = plsc.VectorSubcoreMesh(core_axis_name="c", subcore_axis_name="s")


@functools.partial(
    pl.kernel,
    out_type=jax.ShapeDtypeStruct((NC, NPAD, D), jnp.float32),
    mesh=_mesh,
    scratch_types=[
        [pltpu.VMEM((CHUNK,), jnp.int32) for _ in range(2)],
        pltpu.VMEM((CHUNK, D), jnp.float32),
        pltpu.VMEM_SHARED((NPAD, D), jnp.float32),
        [pltpu.SemaphoreType.DMA for _ in range(2)],
    ],
)
def _deg_kernel(dst_hbm, ones_hbm, zeros_hbm, out_hbm, idx_v, ones_v, deg_sh,
                sem_i):
    cid = lax.axis_index("c")
    sid = lax.axis_index("s")
    base = (cid * NS + sid) * DC
    pltpu.sync_copy(ones_hbm, ones_v)
    pltpu.sync_copy(zeros_hbm, deg_sh.at[pl.ds(sid * RPT, RPT)])
    plsc.subcore_barrier()

    pltpu.async_copy(dst_hbm.at[base + 0], idx_v[0], sem_i[0])
    pltpu.async_copy(dst_hbm.at[base + 1], idx_v[1], sem_i[1])

    def step(i, carry):
        for b in range(2):
            c = 2 * i + b
            pltpu.make_async_copy(dst_hbm.at[base], idx_v[b], sem_i[b]).wait()
            pltpu.sync_copy(ones_v, deg_sh.at[idx_v[b]], add=True)

            @pl.when(c + 2 < DC)
            def _():
                pltpu.async_copy(dst_hbm.at[base + c + 2], idx_v[b], sem_i[b])
        return carry

    lax.fori_loop(0, DC // 2, step, 0)
    pltpu.make_async_copy(dst_hbm.at[base], idx_v[0], sem_i[0]).wait()
    pltpu.sync_copy(ones_v, deg_sh.at[idx_v[0]], add=True)
    plsc.subcore_barrier()
    pltpu.sync_copy(deg_sh.at[pl.ds(sid * RPT, RPT)],
                    out_hbm.at[cid, pl.ds(sid * RPT, RPT)])


@functools.partial(
    pl.kernel,
    out_type=jax.ShapeDtypeStruct((NC, NPAD, D), jnp.float32),
    mesh=_mesh,
    scratch_types=[
        [pltpu.VMEM((CHUNK,), jnp.int32) for _ in range(2)],
        [pltpu.VMEM((CHUNK,), jnp.int32) for _ in range(2)],
        [pltpu.VMEM((CHUNK, D), jnp.float32) for _ in range(2)],
        pltpu.VMEM_SHARED((NPAD, D), jnp.float32),
        [pltpu.SemaphoreType.DMA for _ in range(2)],
        [pltpu.SemaphoreType.DMA for _ in range(2)],
    ],
)
def _agg_kernel(g_hbm, src_hbm, dst_hbm, zeros_hbm, out_hbm,
                src_v, dst_v, rows_v, s_sh, sem_i, sem_g):
    cid = lax.axis_index("c")
    sid = lax.axis_index("s")
    nc = lax.select(cid == 0, AC0, AC1)
    base = lax.select(cid == 0, sid * AC0, NS * AC0 + sid * AC1)
    pltpu.sync_copy(zeros_hbm, s_sh.at[pl.ds(sid * RPT, RPT)])

    def stage(c, b):
        pltpu.async_copy(src_hbm.at[base + c], src_v[b], sem_i[b])
        pltpu.async_copy(dst_hbm.at[base + c], dst_v[b], sem_i[b])

    def wait_stage(b):
        pltpu.make_async_copy(src_hbm.at[base], src_v[b], sem_i[b]).wait()
        pltpu.make_async_copy(dst_hbm.at[base], dst_v[b], sem_i[b]).wait()

    def gather(c, b):
        pltpu.async_copy(g_hbm.at[src_v[b]], rows_v[b], sem_g[b])

    def wait_gather(b):
        pltpu.make_async_copy(g_hbm.at[src_v[b]], rows_v[b], sem_g[b]).wait()

    stage(0, 0)
    stage(1, 1)
    plsc.subcore_barrier()
    wait_stage(0)
    gather(0, 0)

    def step(i, carry):
        for b in range(2):
            c = 2 * i + b
            nb = 1 - b

            @pl.when(c < nc)
            def _():
                @pl.when(c + 1 < nc)
                def _():
                    wait_stage(nb)
                    gather(c + 1, nb)

                wait_gather(b)
                pltpu.sync_copy(rows_v[b], s_sh.at[dst_v[b]], add=True)

                @pl.when(c + 2 < nc)
                def _():
                    stage(c + 2, b)
        return carry

    lax.fori_loop(0, ACMAX // 2, step, 0)
    plsc.subcore_barrier()
    pltpu.sync_copy(s_sh.at[pl.ds(sid * RPT, RPT)],
                    out_hbm.at[cid, pl.ds(sid * RPT, RPT)])


_R = 1280


def _dinv_of(dp):
    deg = 1.0 + dp[0, :, :1] + dp[1, :, :1]
    return lax.rsqrt(deg)


def _pre_body(x_ref, w_ref, dp_ref, h_ref, g_ref):
    dinv = _dinv_of(dp_ref[...])
    h = jnp.dot(x_ref[...], w_ref[...], preferred_element_type=jnp.float32)
    h_ref[...] = h
    g_ref[...] = h * dinv


def _mid_body(s_ref, h_ref, dp_ref, b_ref, w_ref, h2_ref, g2_ref):
    i = pl.program_id(0)
    dinv = _dinv_of(dp_ref[...])
    s = s_ref[0] + s_ref[1]
    pre = dinv * s + dinv * dinv * h_ref[...] + b_ref[...]
    rows = i * _R + lax.broadcasted_iota(jnp.int32, (_R, 1), 0)
    a = jnp.where(rows < N, jnp.maximum(pre, 0.0), 0.0)
    h2 = jnp.dot(a, w_ref[...], preferred_element_type=jnp.float32)
    h2_ref[...] = h2
    g2_ref[...] = h2 * dinv


def _post_body(s_ref, h_ref, dp_ref, b_ref, out_ref):
    dinv = _dinv_of(dp_ref[...])
    s = s_ref[0] + s_ref[1]
    pre = dinv * s + dinv * dinv * h_ref[...] + b_ref[...]
    out_ref[...] = jnp.maximum(pre, 0.0)


_spec_rows = pl.BlockSpec((_R, D), lambda i: (i, 0))
_spec_w = pl.BlockSpec((D, D), lambda i: (0, 0))
_spec_dp = pl.BlockSpec((2, _R, D), lambda i: (0, i, 0))
_spec_s = pl.BlockSpec((2, _R, D), lambda i: (0, i, 0))
_spec_b = pl.BlockSpec((1, D), lambda i: (0, 0))
_grid = (NPAD // _R,)
_f32 = jnp.float32


def _tc_pre(x, w1, dp):
    return pl.pallas_call(
        _pre_body, grid=_grid,
        in_specs=[_spec_rows, _spec_w, _spec_dp],
        out_specs=[_spec_rows, _spec_rows],
        out_shape=[jax.ShapeDtypeStruct((NPAD, D), _f32)] * 2,
    )(x, w1, dp)


def _tc_mid(s, h, dp, b1, w2):
    return pl.pallas_call(
        _mid_body, grid=_grid,
        in_specs=[_spec_s, _spec_rows, _spec_dp, _spec_b, _spec_w],
        out_specs=[_spec_rows, _spec_rows],
        out_shape=[jax.ShapeDtypeStruct((NPAD, D), _f32)] * 2,
    )(s, h, dp, b1, w2)


def _tc_post(s, h, dp, b2):
    return pl.pallas_call(
        _post_body, grid=_grid,
        in_specs=[_spec_s, _spec_rows, _spec_dp, _spec_b],
        out_specs=_spec_rows,
        out_shape=jax.ShapeDtypeStruct((NPAD, D), _f32),
    )(s, h, dp, b2)


def kernel(x, edge_index, W1, b1, W2, b2):
    src = edge_index[0].astype(jnp.int32)
    dst = edge_index[1].astype(jnp.int32)
    pad = jnp.full((EPAD - E,), N, dtype=jnp.int32)
    src_r = jnp.concatenate([src, pad]).reshape(CT, CHUNK)
    dst_r = jnp.concatenate([dst, pad]).reshape(CT, CHUNK)

    x_pad = jnp.pad(x, ((0, NPAD - N), (0, 0)))
    ones128 = jnp.ones((CHUNK, D), jnp.float32)
    zeros128 = jnp.zeros((RPT, D), jnp.float32)
    b1r = b1.reshape(1, D)
    b2r = b2.reshape(1, D)

    dp = _deg_kernel(dst_r, ones128, zeros128)
    h1, g1 = _tc_pre(x_pad, W1, dp)
    s1 = _agg_kernel(g1, src_r, dst_r, zeros128)
    h2, g2 = _tc_mid(s1, h1, dp, b1r, W2)
    s2 = _agg_kernel(g2, src_r, dst_r, zeros128)
    out = _tc_post(s2, h2, dp, b2r)
    return (out[:N], edge_index)


# speedup vs baseline: 4.0099x; 1.2603x over previous
import functools

import jax
import jax.numpy as jnp
from jax import lax
from jax.experimental import pallas as pl
from jax.experimental.pallas import tpu as pltpu
from jax.experimental.pallas import tpu_sc as plsc

N = 10000
E = 320000
D = 128

NC = 2
NS = 16
NW = NC * NS

NPAD = 10240
RPT = NPAD // NS
CHUNK = 128
CT = 2528
EPAD = CT * CHUNK

AC0 = 118
AC1 = 40
ACMAX = 118

DC = CT // NW

_mesh = plsc.VectorSubcoreMesh(core_axis_name="c", subcore_axis_name="s")


@functools.partial(
    pl.kernel,
    out_type=jax.ShapeDtypeStruct((NC, NPAD, D), jnp.float32),
    mesh=_mesh,
    scratch_types=[
        [pltpu.VMEM((CHUNK,), jnp.int32) for _ in range(2)],
        pltpu.VMEM((CHUNK, D), jnp.float32),
        pltpu.VMEM_SHARED((NPAD, D), jnp.float32),
        [pltpu.SemaphoreType.DMA for _ in range(2)],
    ],
)
def _deg_kernel(dst_hbm, ones_hbm, zeros_hbm, out_hbm, idx_v, ones_v, deg_sh,
                sem_i):
    cid = lax.axis_index("c")
    sid = lax.axis_index("s")
    base = (cid * NS + sid) * DC
    pltpu.sync_copy(ones_hbm, ones_v)
    pltpu.sync_copy(zeros_hbm, deg_sh.at[pl.ds(sid * RPT, RPT)])
    plsc.subcore_barrier()

    pltpu.async_copy(dst_hbm.at[base + 0], idx_v[0], sem_i[0])
    pltpu.async_copy(dst_hbm.at[base + 1], idx_v[1], sem_i[1])

    def step(i, carry):
        for b in range(2):
            c = 2 * i + b
            pltpu.make_async_copy(dst_hbm.at[base], idx_v[b], sem_i[b]).wait()
            pltpu.sync_copy(ones_v, deg_sh.at[idx_v[b]], add=True)

            @pl.when(c + 2 < DC)
            def _():
                pltpu.async_copy(dst_hbm.at[base + c + 2], idx_v[b], sem_i[b])
        return carry

    lax.fori_loop(0, DC // 2, step, 0)
    pltpu.make_async_copy(dst_hbm.at[base], idx_v[0], sem_i[0]).wait()
    pltpu.sync_copy(ones_v, deg_sh.at[idx_v[0]], add=True)
    plsc.subcore_barrier()
    pltpu.sync_copy(deg_sh.at[pl.ds(sid * RPT, RPT)],
                    out_hbm.at[cid, pl.ds(sid * RPT, RPT)])


@functools.partial(
    pl.kernel,
    out_type=jax.ShapeDtypeStruct((NC, NPAD, D), jnp.float32),
    mesh=_mesh,
    scratch_types=[
        [pltpu.VMEM((CHUNK,), jnp.int32) for _ in range(2)],
        [pltpu.VMEM((CHUNK,), jnp.int32) for _ in range(2)],
        [pltpu.VMEM((CHUNK, D), jnp.float32) for _ in range(2)],
        pltpu.VMEM_SHARED((NPAD, D), jnp.float32),
        [pltpu.SemaphoreType.DMA for _ in range(2)],
        [pltpu.SemaphoreType.DMA for _ in range(2)],
    ],
)
def _agg_kernel(g_hbm, src_hbm, dst_hbm, zeros_hbm, out_hbm,
                src_v, dst_v, rows_v, s_sh, sem_i, sem_g):
    cid = lax.axis_index("c")
    sid = lax.axis_index("s")
    nc = lax.select(cid == 0, AC0, AC1)
    base = lax.select(cid == 0, sid * AC0, NS * AC0 + sid * AC1)
    pltpu.sync_copy(zeros_hbm, s_sh.at[pl.ds(sid * RPT, RPT)])

    def stage(c, b):
        pltpu.async_copy(src_hbm.at[base + c], src_v[b], sem_i[b])
        pltpu.async_copy(dst_hbm.at[base + c], dst_v[b], sem_i[b])

    def wait_stage(b):
        pltpu.make_async_copy(src_hbm.at[base], src_v[b], sem_i[b]).wait()
        pltpu.make_async_copy(dst_hbm.at[base], dst_v[b], sem_i[b]).wait()

    def gather(c, b):
        pltpu.async_copy(g_hbm.at[src_v[b]], rows_v[b], sem_g[b])

    def wait_gather(b):
        pltpu.make_async_copy(g_hbm.at[src_v[b]], rows_v[b], sem_g[b]).wait()

    stage(0, 0)
    stage(1, 1)
    plsc.subcore_barrier()
    wait_stage(0)
    gather(0, 0)

    def step(i, carry):
        for b in range(2):
            c = 2 * i + b
            nb = 1 - b

            @pl.when(c < nc)
            def _():
                @pl.when(c + 1 < nc)
                def _():
                    wait_stage(nb)
                    gather(c + 1, nb)

                wait_gather(b)
                pltpu.sync_copy(rows_v[b], s_sh.at[dst_v[b]], add=True)

                @pl.when(c + 2 < nc)
                def _():
                    stage(c + 2, b)
        return carry

    lax.fori_loop(0, ACMAX // 2, step, 0)
    plsc.subcore_barrier()
    pltpu.sync_copy(s_sh.at[pl.ds(sid * RPT, RPT)],
                    out_hbm.at[cid, pl.ds(sid * RPT, RPT)])


_R = 1280


def _dinv_of(dp):
    deg = 1.0 + dp[0, :, :1] + dp[1, :, :1]
    return lax.rsqrt(deg)


def _pre_body(x_ref, w_ref, dp_ref, h_ref, g_ref):
    dinv = _dinv_of(dp_ref[...])
    h = jnp.dot(x_ref[...], w_ref[...], preferred_element_type=jnp.float32)
    h_ref[...] = h
    g_ref[...] = h * dinv


def _mid_body(s_ref, h_ref, dp_ref, b_ref, w_ref, h2_ref, g2_ref):
    i = pl.program_id(0)
    dinv = _dinv_of(dp_ref[...])
    s = s_ref[0] + s_ref[1]
    pre = dinv * s + dinv * dinv * h_ref[...] + b_ref[...]
    rows = i * _R + lax.broadcasted_iota(jnp.int32, (_R, 1), 0)
    a = jnp.where(rows < N, jnp.maximum(pre, 0.0), 0.0)
    h2 = jnp.dot(a, w_ref[...], preferred_element_type=jnp.float32)
    h2_ref[...] = h2
    g2_ref[...] = h2 * dinv


def _post_body(s_ref, h_ref, dp_ref, b_ref, out_ref):
    dinv = _dinv_of(dp_ref[...])
    s = s_ref[0] + s_ref[1]
    pre = dinv * s + dinv * dinv * h_ref[...] + b_ref[...]
    out_ref[...] = jnp.maximum(pre, 0.0)


_spec_rows = pl.BlockSpec((_R, D), lambda i: (i, 0))
_spec_w = pl.BlockSpec((D, D), lambda i: (0, 0))
_spec_dp = pl.BlockSpec((2, _R, D), lambda i: (0, i, 0))
_spec_s = pl.BlockSpec((2, _R, D), lambda i: (0, i, 0))
_spec_b = pl.BlockSpec((1, D), lambda i: (0, 0))
_grid = (NPAD // _R,)
_f32 = jnp.float32


def _tc_pre(x, w1, dp):
    return pl.pallas_call(
        _pre_body, grid=_grid,
        in_specs=[_spec_rows, _spec_w, _spec_dp],
        out_specs=[_spec_rows, _spec_rows],
        out_shape=[jax.ShapeDtypeStruct((NPAD, D), _f32)] * 2,
    )(x, w1, dp)


def _tc_mid(s, h, dp, b1, w2):
    return pl.pallas_call(
        _mid_body, grid=_grid,
        in_specs=[_spec_s, _spec_rows, _spec_dp, _spec_b, _spec_w],
        out_specs=[_spec_rows, _spec_rows],
        out_shape=[jax.ShapeDtypeStruct((NPAD, D), _f32)] * 2,
    )(s, h, dp, b1, w2)


def _tc_post(s, h, dp, b2):
    return pl.pallas_call(
        _post_body, grid=_grid,
        in_specs=[_spec_s, _spec_rows, _spec_dp, _spec_b],
        out_specs=_spec_rows,
        out_shape=jax.ShapeDtypeStruct((NPAD, D), _f32),
    )(s, h, dp, b2)


def kernel(x, edge_index, W1, b1, W2, b2):
    src = edge_index[0].astype(jnp.int32)
    dst = edge_index[1].astype(jnp.int32)
    pad = jnp.full((EPAD - E,), N, dtype=jnp.int32)
    src_r = jnp.concatenate([src, pad]).reshape(CT, CHUNK)
    dst_r = jnp.concatenate([dst, pad]).reshape(CT, CHUNK)

    x_pad = jnp.pad(x, ((0, NPAD - N), (0, 0)))
    ones128 = jnp.ones((CHUNK, D), jnp.float32)
    zeros128 = jnp.zeros((RPT, D), jnp.float32)
    b1r = b1.reshape(1, D)
    b2r = b2.reshape(1, D)

    dp = _deg_kernel(dst_r, ones128, zeros128)
    h1, g1 = _tc_pre(x_pad, W1, dp)
    s1 = _agg_kernel(g1, src_r, dst_r, zeros128)
    h2, g2 = _tc_mid(s1, h1, dp, b1r, W2)
    s2 = _agg_kernel(g2, src_r, dst_r, zeros128)
    out = _tc_post(s2, h2, dp, b2r)
    return (out[:N], edge_index)


# speedup vs baseline: 4.1301x; 1.0300x over previous
import functools

import jax
import jax.numpy as jnp
from jax import lax
from jax.experimental import pallas as pl
from jax.experimental.pallas import tpu as pltpu
from jax.experimental.pallas import tpu_sc as plsc

N = 10000
E = 320000
D = 128

NC = 2
NS = 16
NW = NC * NS

NPAD = 10240
RPT = NPAD // NS
CHUNK = 128
CT = 2528
EPAD = CT * CHUNK

AC0 = 126
AC1 = 32
ACMAX = 126

DC = CT // NW

_mesh = plsc.VectorSubcoreMesh(core_axis_name="c", subcore_axis_name="s")


@functools.partial(
    pl.kernel,
    out_type=jax.ShapeDtypeStruct((NC, NPAD, D), jnp.float32),
    mesh=_mesh,
    scratch_types=[
        [pltpu.VMEM((CHUNK,), jnp.int32) for _ in range(2)],
        pltpu.VMEM((CHUNK, D), jnp.float32),
        pltpu.VMEM_SHARED((NPAD, D), jnp.float32),
        [pltpu.SemaphoreType.DMA for _ in range(2)],
    ],
)
def _deg_kernel(dst_hbm, ones_hbm, zeros_hbm, out_hbm, idx_v, ones_v, deg_sh,
                sem_i):
    cid = lax.axis_index("c")
    sid = lax.axis_index("s")
    base = (cid * NS + sid) * DC
    pltpu.sync_copy(ones_hbm, ones_v)
    pltpu.sync_copy(zeros_hbm, deg_sh.at[pl.ds(sid * RPT, RPT)])
    plsc.subcore_barrier()

    pltpu.async_copy(dst_hbm.at[base + 0], idx_v[0], sem_i[0])
    pltpu.async_copy(dst_hbm.at[base + 1], idx_v[1], sem_i[1])

    def step(i, carry):
        for b in range(2):
            c = 2 * i + b
            pltpu.make_async_copy(dst_hbm.at[base], idx_v[b], sem_i[b]).wait()
            pltpu.sync_copy(ones_v, deg_sh.at[idx_v[b]], add=True)

            @pl.when(c + 2 < DC)
            def _():
                pltpu.async_copy(dst_hbm.at[base + c + 2], idx_v[b], sem_i[b])
        return carry

    lax.fori_loop(0, DC // 2, step, 0)
    pltpu.make_async_copy(dst_hbm.at[base], idx_v[0], sem_i[0]).wait()
    pltpu.sync_copy(ones_v, deg_sh.at[idx_v[0]], add=True)
    plsc.subcore_barrier()
    pltpu.sync_copy(deg_sh.at[pl.ds(sid * RPT, RPT)],
                    out_hbm.at[cid, pl.ds(sid * RPT, RPT)])


@functools.partial(
    pl.kernel,
    out_type=jax.ShapeDtypeStruct((NC, NPAD, D), jnp.float32),
    mesh=_mesh,
    scratch_types=[
        [pltpu.VMEM((CHUNK,), jnp.int32) for _ in range(2)],
        [pltpu.VMEM((CHUNK,), jnp.int32) for _ in range(2)],
        [pltpu.VMEM((CHUNK, D), jnp.float32) for _ in range(2)],
        pltpu.VMEM_SHARED((NPAD, D), jnp.float32),
        [pltpu.SemaphoreType.DMA for _ in range(2)],
        [pltpu.SemaphoreType.DMA for _ in range(2)],
    ],
)
def _agg_kernel(g_hbm, src_hbm, dst_hbm, zeros_hbm, out_hbm,
                src_v, dst_v, rows_v, s_sh, sem_i, sem_g):
    cid = lax.axis_index("c")
    sid = lax.axis_index("s")
    nc = lax.select(cid == 0, AC0, AC1)
    base = lax.select(cid == 0, sid * AC0, NS * AC0 + sid * AC1)
    pltpu.sync_copy(zeros_hbm, s_sh.at[pl.ds(sid * RPT, RPT)])

    def stage(c, b):
        pltpu.async_copy(src_hbm.at[base + c], src_v[b], sem_i[b])
        pltpu.async_copy(dst_hbm.at[base + c], dst_v[b], sem_i[b])

    def wait_stage(b):
        pltpu.make_async_copy(src_hbm.at[base], src_v[b], sem_i[b]).wait()
        pltpu.make_async_copy(dst_hbm.at[base], dst_v[b], sem_i[b]).wait()

    def gather(c, b):
        pltpu.async_copy(g_hbm.at[src_v[b]], rows_v[b], sem_g[b])

    def wait_gather(b):
        pltpu.make_async_copy(g_hbm.at[src_v[b]], rows_v[b], sem_g[b]).wait()

    stage(0, 0)
    stage(1, 1)
    plsc.subcore_barrier()
    wait_stage(0)
    gather(0, 0)

    def step(i, carry):
        for b in range(2):
            c = 2 * i + b
            nb = 1 - b

            @pl.when(c < nc)
            def _():
                @pl.when(c + 1 < nc)
                def _():
                    wait_stage(nb)
                    gather(c + 1, nb)

                wait_gather(b)
                pltpu.sync_copy(rows_v[b], s_sh.at[dst_v[b]], add=True)

                @pl.when(c + 2 < nc)
                def _():
                    stage(c + 2, b)
        return carry

    lax.fori_loop(0, ACMAX // 2, step, 0)
    plsc.subcore_barrier()
    pltpu.sync_copy(s_sh.at[pl.ds(sid * RPT, RPT)],
                    out_hbm.at[cid, pl.ds(sid * RPT, RPT)])


_R = 1280


def _dinv_of(dp):
    deg = 1.0 + dp[0, :, :1] + dp[1, :, :1]
    return lax.rsqrt(deg)


def _pre_body(x_ref, w_ref, dp_ref, h_ref, g_ref):
    dinv = _dinv_of(dp_ref[...])
    h = jnp.dot(x_ref[...], w_ref[...], preferred_element_type=jnp.float32)
    h_ref[...] = h
    g_ref[...] = h * dinv


def _mid_body(s_ref, h_ref, dp_ref, b_ref, w_ref, h2_ref, g2_ref):
    i = pl.program_id(0)
    dinv = _dinv_of(dp_ref[...])
    s = s_ref[0] + s_ref[1]
    pre = dinv * s + dinv * dinv * h_ref[...] + b_ref[...]
    rows = i * _R + lax.broadcasted_iota(jnp.int32, (_R, 1), 0)
    a = jnp.where(rows < N, jnp.maximum(pre, 0.0), 0.0)
    h2 = jnp.dot(a, w_ref[...], preferred_element_type=jnp.float32)
    h2_ref[...] = h2
    g2_ref[...] = h2 * dinv


def _post_body(s_ref, h_ref, dp_ref, b_ref, out_ref):
    dinv = _dinv_of(dp_ref[...])
    s = s_ref[0] + s_ref[1]
    pre = dinv * s + dinv * dinv * h_ref[...] + b_ref[...]
    out_ref[...] = jnp.maximum(pre, 0.0)


_spec_rows = pl.BlockSpec((_R, D), lambda i: (i, 0))
_spec_w = pl.BlockSpec((D, D), lambda i: (0, 0))
_spec_dp = pl.BlockSpec((2, _R, D), lambda i: (0, i, 0))
_spec_s = pl.BlockSpec((2, _R, D), lambda i: (0, i, 0))
_spec_b = pl.BlockSpec((1, D), lambda i: (0, 0))
_grid = (NPAD // _R,)
_f32 = jnp.float32


def _tc_pre(x, w1, dp):
    return pl.pallas_call(
        _pre_body, grid=_grid,
        in_specs=[_spec_rows, _spec_w, _spec_dp],
        out_specs=[_spec_rows, _spec_rows],
        out_shape=[jax.ShapeDtypeStruct((NPAD, D), _f32)] * 2,
    )(x, w1, dp)


def _tc_mid(s, h, dp, b1, w2):
    return pl.pallas_call(
        _mid_body, grid=_grid,
        in_specs=[_spec_s, _spec_rows, _spec_dp, _spec_b, _spec_w],
        out_specs=[_spec_rows, _spec_rows],
        out_shape=[jax.ShapeDtypeStruct((NPAD, D), _f32)] * 2,
    )(s, h, dp, b1, w2)


def _tc_post(s, h, dp, b2):
    return pl.pallas_call(
        _post_body, grid=_grid,
        in_specs=[_spec_s, _spec_rows, _spec_dp, _spec_b],
        out_specs=_spec_rows,
        out_shape=jax.ShapeDtypeStruct((NPAD, D), _f32),
    )(s, h, dp, b2)


def kernel(x, edge_index, W1, b1, W2, b2):
    src = edge_index[0].astype(jnp.int32)
    dst = edge_index[1].astype(jnp.int32)
    pad = jnp.full((EPAD - E,), N, dtype=jnp.int32)
    src_r = jnp.concatenate([src, pad]).reshape(CT, CHUNK)
    dst_r = jnp.concatenate([dst, pad]).reshape(CT, CHUNK)

    x_pad = jnp.pad(x, ((0, NPAD - N), (0, 0)))
    ones128 = jnp.ones((CHUNK, D), jnp.float32)
    zeros128 = jnp.zeros((RPT, D), jnp.float32)
    b1r = b1.reshape(1, D)
    b2r = b2.reshape(1, D)

    dp = _deg_kernel(dst_r, ones128, zeros128)
    h1, g1 = _tc_pre(x_pad, W1, dp)
    s1 = _agg_kernel(g1, src_r, dst_r, zeros128)
    h2, g2 = _tc_mid(s1, h1, dp, b1r, W2)
    s2 = _agg_kernel(g2, src_r, dst_r, zeros128)
    out = _tc_post(s2, h2, dp, b2r)
    return (out[:N], edge_index)


# speedup vs baseline: 4.2045x; 1.0180x over previous
import functools

import jax
import jax.numpy as jnp
from jax import lax
from jax.experimental import pallas as pl
from jax.experimental.pallas import tpu as pltpu
from jax.experimental.pallas import tpu_sc as plsc

N = 10000
E = 320000
D = 128

NC = 2
NS = 16
NW = NC * NS

NPAD = 10240
RPT = NPAD // NS
CHUNK = 128
CT = 2528
EPAD = CT * CHUNK

AC0 = 130
AC1 = 28
ACMAX = 130

DC = CT // NW

_mesh = plsc.VectorSubcoreMesh(core_axis_name="c", subcore_axis_name="s")


@functools.partial(
    pl.kernel,
    out_type=jax.ShapeDtypeStruct((NC, NPAD, D), jnp.float32),
    mesh=_mesh,
    scratch_types=[
        [pltpu.VMEM((CHUNK,), jnp.int32) for _ in range(2)],
        pltpu.VMEM((CHUNK, D), jnp.float32),
        pltpu.VMEM_SHARED((NPAD, D), jnp.float32),
        [pltpu.SemaphoreType.DMA for _ in range(2)],
    ],
)
def _deg_kernel(dst_hbm, ones_hbm, zeros_hbm, out_hbm, idx_v, ones_v, deg_sh,
                sem_i):
    cid = lax.axis_index("c")
    sid = lax.axis_index("s")
    base = (cid * NS + sid) * DC
    pltpu.sync_copy(ones_hbm, ones_v)
    pltpu.sync_copy(zeros_hbm, deg_sh.at[pl.ds(sid * RPT, RPT)])
    plsc.subcore_barrier()

    pltpu.async_copy(dst_hbm.at[base + 0], idx_v[0], sem_i[0])
    pltpu.async_copy(dst_hbm.at[base + 1], idx_v[1], sem_i[1])

    def step(i, carry):
        for b in range(2):
            c = 2 * i + b
            pltpu.make_async_copy(dst_hbm.at[base], idx_v[b], sem_i[b]).wait()
            pltpu.sync_copy(ones_v, deg_sh.at[idx_v[b]], add=True)

            @pl.when(c + 2 < DC)
            def _():
                pltpu.async_copy(dst_hbm.at[base + c + 2], idx_v[b], sem_i[b])
        return carry

    lax.fori_loop(0, DC // 2, step, 0)
    pltpu.make_async_copy(dst_hbm.at[base], idx_v[0], sem_i[0]).wait()
    pltpu.sync_copy(ones_v, deg_sh.at[idx_v[0]], add=True)
    plsc.subcore_barrier()
    pltpu.sync_copy(deg_sh.at[pl.ds(sid * RPT, RPT)],
                    out_hbm.at[cid, pl.ds(sid * RPT, RPT)])


@functools.partial(
    pl.kernel,
    out_type=jax.ShapeDtypeStruct((NC, NPAD, D), jnp.float32),
    mesh=_mesh,
    scratch_types=[
        [pltpu.VMEM((CHUNK,), jnp.int32) for _ in range(2)],
        [pltpu.VMEM((CHUNK,), jnp.int32) for _ in range(2)],
        [pltpu.VMEM((CHUNK, D), jnp.float32) for _ in range(2)],
        pltpu.VMEM_SHARED((NPAD, D), jnp.float32),
        [pltpu.SemaphoreType.DMA for _ in range(2)],
        [pltpu.SemaphoreType.DMA for _ in range(2)],
    ],
)
def _agg_kernel(g_hbm, src_hbm, dst_hbm, zeros_hbm, out_hbm,
                src_v, dst_v, rows_v, s_sh, sem_i, sem_g):
    cid = lax.axis_index("c")
    sid = lax.axis_index("s")
    nc = lax.select(cid == 0, AC0, AC1)
    base = lax.select(cid == 0, sid * AC0, NS * AC0 + sid * AC1)
    pltpu.sync_copy(zeros_hbm, s_sh.at[pl.ds(sid * RPT, RPT)])

    def stage(c, b):
        pltpu.async_copy(src_hbm.at[base + c], src_v[b], sem_i[b])
        pltpu.async_copy(dst_hbm.at[base + c], dst_v[b], sem_i[b])

    def wait_stage(b):
        pltpu.make_async_copy(src_hbm.at[base], src_v[b], sem_i[b]).wait()
        pltpu.make_async_copy(dst_hbm.at[base], dst_v[b], sem_i[b]).wait()

    def gather(c, b):
        pltpu.async_copy(g_hbm.at[src_v[b]], rows_v[b], sem_g[b])

    def wait_gather(b):
        pltpu.make_async_copy(g_hbm.at[src_v[b]], rows_v[b], sem_g[b]).wait()

    stage(0, 0)
    stage(1, 1)
    plsc.subcore_barrier()
    wait_stage(0)
    gather(0, 0)

    def step(i, carry):
        for b in range(2):
            c = 2 * i + b
            nb = 1 - b

            @pl.when(c < nc)
            def _():
                @pl.when(c + 1 < nc)
                def _():
                    wait_stage(nb)
                    gather(c + 1, nb)

                wait_gather(b)
                pltpu.sync_copy(rows_v[b], s_sh.at[dst_v[b]], add=True)

                @pl.when(c + 2 < nc)
                def _():
                    stage(c + 2, b)
        return carry

    lax.fori_loop(0, ACMAX // 2, step, 0)
    plsc.subcore_barrier()
    pltpu.sync_copy(s_sh.at[pl.ds(sid * RPT, RPT)],
                    out_hbm.at[cid, pl.ds(sid * RPT, RPT)])


_R = 1280


def _dinv_of(dp):
    deg = 1.0 + dp[0, :, :1] + dp[1, :, :1]
    return lax.rsqrt(deg)


def _pre_body(x_ref, w_ref, dp_ref, h_ref, g_ref):
    dinv = _dinv_of(dp_ref[...])
    h = jnp.dot(x_ref[...], w_ref[...], preferred_element_type=jnp.float32)
    h_ref[...] = h
    g_ref[...] = h * dinv


def _mid_body(s_ref, h_ref, dp_ref, b_ref, w_ref, h2_ref, g2_ref):
    i = pl.program_id(0)
    dinv = _dinv_of(dp_ref[...])
    s = s_ref[0] + s_ref[1]
    pre = dinv * s + dinv * dinv * h_ref[...] + b_ref[...]
    rows = i * _R + lax.broadcasted_iota(jnp.int32, (_R, 1), 0)
    a = jnp.where(rows < N, jnp.maximum(pre, 0.0), 0.0)
    h2 = jnp.dot(a, w_ref[...], preferred_element_type=jnp.float32)
    h2_ref[...] = h2
    g2_ref[...] = h2 * dinv


def _post_body(s_ref, h_ref, dp_ref, b_ref, out_ref):
    dinv = _dinv_of(dp_ref[...])
    s = s_ref[0] + s_ref[1]
    pre = dinv * s + dinv * dinv * h_ref[...] + b_ref[...]
    out_ref[...] = jnp.maximum(pre, 0.0)


_spec_rows = pl.BlockSpec((_R, D), lambda i: (i, 0))
_spec_w = pl.BlockSpec((D, D), lambda i: (0, 0))
_spec_dp = pl.BlockSpec((2, _R, D), lambda i: (0, i, 0))
_spec_s = pl.BlockSpec((2, _R, D), lambda i: (0, i, 0))
_spec_b = pl.BlockSpec((1, D), lambda i: (0, 0))
_grid = (NPAD // _R,)
_f32 = jnp.float32


def _tc_pre(x, w1, dp):
    return pl.pallas_call(
        _pre_body, grid=_grid,
        in_specs=[_spec_rows, _spec_w, _spec_dp],
        out_specs=[_spec_rows, _spec_rows],
        out_shape=[jax.ShapeDtypeStruct((NPAD, D), _f32)] * 2,
    )(x, w1, dp)


def _tc_mid(s, h, dp, b1, w2):
    return pl.pallas_call(
        _mid_body, grid=_grid,
        in_specs=[_spec_s, _spec_rows, _spec_dp, _spec_b, _spec_w],
        out_specs=[_spec_rows, _spec_rows],
        out_shape=[jax.ShapeDtypeStruct((NPAD, D), _f32)] * 2,
    )(s, h, dp, b1, w2)


def _tc_post(s, h, dp, b2):
    return pl.pallas_call(
        _post_body, grid=_grid,
        in_specs=[_spec_s, _spec_rows, _spec_dp, _spec_b],
        out_specs=_spec_rows,
        out_shape=jax.ShapeDtypeStruct((NPAD, D), _f32),
    )(s, h, dp, b2)


def kernel(x, edge_index, W1, b1, W2, b2):
    src = edge_index[0].astype(jnp.int32)
    dst = edge_index[1].astype(jnp.int32)
    pad = jnp.full((EPAD - E,), N, dtype=jnp.int32)
    src_r = jnp.concatenate([src, pad]).reshape(CT, CHUNK)
    dst_r = jnp.concatenate([dst, pad]).reshape(CT, CHUNK)

    x_pad = jnp.pad(x, ((0, NPAD - N), (0, 0)))
    ones128 = jnp.ones((CHUNK, D), jnp.float32)
    zeros128 = jnp.zeros((RPT, D), jnp.float32)
    b1r = b1.reshape(1, D)
    b2r = b2.reshape(1, D)

    dp = _deg_kernel(dst_r, ones128, zeros128)
    h1, g1 = _tc_pre(x_pad, W1, dp)
    s1 = _agg_kernel(g1, src_r, dst_r, zeros128)
    h2, g2 = _tc_mid(s1, h1, dp, b1r, W2)
    s2 = _agg_kernel(g2, src_r, dst_r, zeros128)
    out = _tc_post(s2, h2, dp, b2r)
    return (out[:N], edge_index)


# speedup vs baseline: 4.2247x; 1.0048x over previous
import functools

import jax
import jax.numpy as jnp
from jax import lax
from jax.experimental import pallas as pl
from jax.experimental.pallas import tpu as pltpu
from jax.experimental.pallas import tpu_sc as plsc

N = 10000
E = 320000
D = 128

NC = 2
NS = 16
NW = NC * NS

NPAD = 10240
RPT = NPAD // NS
CHUNK = 128
CT = 2528
EPAD = CT * CHUNK

AC0 = 134
AC1 = 24
ACMAX = 134

DC = CT // NW

_mesh = plsc.VectorSubcoreMesh(core_axis_name="c", subcore_axis_name="s")


@functools.partial(
    pl.kernel,
    out_type=jax.ShapeDtypeStruct((NC, NPAD, D), jnp.float32),
    mesh=_mesh,
    scratch_types=[
        [pltpu.VMEM((CHUNK,), jnp.int32) for _ in range(2)],
        pltpu.VMEM((CHUNK, D), jnp.float32),
        pltpu.VMEM_SHARED((NPAD, D), jnp.float32),
        [pltpu.SemaphoreType.DMA for _ in range(2)],
    ],
)
def _deg_kernel(dst_hbm, ones_hbm, zeros_hbm, out_hbm, idx_v, ones_v, deg_sh,
                sem_i):
    cid = lax.axis_index("c")
    sid = lax.axis_index("s")
    base = (cid * NS + sid) * DC
    pltpu.sync_copy(ones_hbm, ones_v)
    pltpu.sync_copy(zeros_hbm, deg_sh.at[pl.ds(sid * RPT, RPT)])
    plsc.subcore_barrier()

    pltpu.async_copy(dst_hbm.at[base + 0], idx_v[0], sem_i[0])
    pltpu.async_copy(dst_hbm.at[base + 1], idx_v[1], sem_i[1])

    def step(i, carry):
        for b in range(2):
            c = 2 * i + b
            pltpu.make_async_copy(dst_hbm.at[base], idx_v[b], sem_i[b]).wait()
            pltpu.sync_copy(ones_v, deg_sh.at[idx_v[b]], add=True)

            @pl.when(c + 2 < DC)
            def _():
                pltpu.async_copy(dst_hbm.at[base + c + 2], idx_v[b], sem_i[b])
        return carry

    lax.fori_loop(0, DC // 2, step, 0)
    pltpu.make_async_copy(dst_hbm.at[base], idx_v[0], sem_i[0]).wait()
    pltpu.sync_copy(ones_v, deg_sh.at[idx_v[0]], add=True)
    plsc.subcore_barrier()
    pltpu.sync_copy(deg_sh.at[pl.ds(sid * RPT, RPT)],
                    out_hbm.at[cid, pl.ds(sid * RPT, RPT)])


@functools.partial(
    pl.kernel,
    out_type=jax.ShapeDtypeStruct((NC, NPAD, D), jnp.float32),
    mesh=_mesh,
    scratch_types=[
        [pltpu.VMEM((CHUNK,), jnp.int32) for _ in range(2)],
        [pltpu.VMEM((CHUNK,), jnp.int32) for _ in range(2)],
        [pltpu.VMEM((CHUNK, D), jnp.float32) for _ in range(2)],
        pltpu.VMEM_SHARED((NPAD, D), jnp.float32),
        [pltpu.SemaphoreType.DMA for _ in range(2)],
        [pltpu.SemaphoreType.DMA for _ in range(2)],
    ],
)
def _agg_kernel(g_hbm, src_hbm, dst_hbm, zeros_hbm, out_hbm,
                src_v, dst_v, rows_v, s_sh, sem_i, sem_g):
    cid = lax.axis_index("c")
    sid = lax.axis_index("s")
    nc = lax.select(cid == 0, AC0, AC1)
    base = lax.select(cid == 0, sid * AC0, NS * AC0 + sid * AC1)
    pltpu.sync_copy(zeros_hbm, s_sh.at[pl.ds(sid * RPT, RPT)])

    def stage(c, b):
        pltpu.async_copy(src_hbm.at[base + c], src_v[b], sem_i[b])
        pltpu.async_copy(dst_hbm.at[base + c], dst_v[b], sem_i[b])

    def wait_stage(b):
        pltpu.make_async_copy(src_hbm.at[base], src_v[b], sem_i[b]).wait()
        pltpu.make_async_copy(dst_hbm.at[base], dst_v[b], sem_i[b]).wait()

    def gather(c, b):
        pltpu.async_copy(g_hbm.at[src_v[b]], rows_v[b], sem_g[b])

    def wait_gather(b):
        pltpu.make_async_copy(g_hbm.at[src_v[b]], rows_v[b], sem_g[b]).wait()

    stage(0, 0)
    stage(1, 1)
    plsc.subcore_barrier()
    wait_stage(0)
    gather(0, 0)

    def step(i, carry):
        for b in range(2):
            c = 2 * i + b
            nb = 1 - b

            @pl.when(c < nc)
            def _():
                @pl.when(c + 1 < nc)
                def _():
                    wait_stage(nb)
                    gather(c + 1, nb)

                wait_gather(b)
                pltpu.sync_copy(rows_v[b], s_sh.at[dst_v[b]], add=True)

                @pl.when(c + 2 < nc)
                def _():
                    stage(c + 2, b)
        return carry

    lax.fori_loop(0, ACMAX // 2, step, 0)
    plsc.subcore_barrier()
    pltpu.sync_copy(s_sh.at[pl.ds(sid * RPT, RPT)],
                    out_hbm.at[cid, pl.ds(sid * RPT, RPT)])


_R = 1280


def _dinv_of(dp):
    deg = 1.0 + dp[0, :, :1] + dp[1, :, :1]
    return lax.rsqrt(deg)


def _pre_body(x_ref, w_ref, dp_ref, h_ref, g_ref):
    dinv = _dinv_of(dp_ref[...])
    h = jnp.dot(x_ref[...], w_ref[...], preferred_element_type=jnp.float32)
    h_ref[...] = h
    g_ref[...] = h * dinv


def _mid_body(s_ref, h_ref, dp_ref, b_ref, w_ref, h2_ref, g2_ref):
    i = pl.program_id(0)
    dinv = _dinv_of(dp_ref[...])
    s = s_ref[0] + s_ref[1]
    pre = dinv * s + dinv * dinv * h_ref[...] + b_ref[...]
    rows = i * _R + lax.broadcasted_iota(jnp.int32, (_R, 1), 0)
    a = jnp.where(rows < N, jnp.maximum(pre, 0.0), 0.0)
    h2 = jnp.dot(a, w_ref[...], preferred_element_type=jnp.float32)
    h2_ref[...] = h2
    g2_ref[...] = h2 * dinv


def _post_body(s_ref, h_ref, dp_ref, b_ref, out_ref):
    dinv = _dinv_of(dp_ref[...])
    s = s_ref[0] + s_ref[1]
    pre = dinv * s + dinv * dinv * h_ref[...] + b_ref[...]
    out_ref[...] = jnp.maximum(pre, 0.0)


_spec_rows = pl.BlockSpec((_R, D), lambda i: (i, 0))
_spec_w = pl.BlockSpec((D, D), lambda i: (0, 0))
_spec_dp = pl.BlockSpec((2, _R, D), lambda i: (0, i, 0))
_spec_s = pl.BlockSpec((2, _R, D), lambda i: (0, i, 0))
_spec_b = pl.BlockSpec((1, D), lambda i: (0, 0))
_grid = (NPAD // _R,)
_f32 = jnp.float32


def _tc_pre(x, w1, dp):
    return pl.pallas_call(
        _pre_body, grid=_grid,
        in_specs=[_spec_rows, _spec_w, _spec_dp],
        out_specs=[_spec_rows, _spec_rows],
        out_shape=[jax.ShapeDtypeStruct((NPAD, D), _f32)] * 2,
    )(x, w1, dp)


def _tc_mid(s, h, dp, b1, w2):
    return pl.pallas_call(
        _mid_body, grid=_grid,
        in_specs=[_spec_s, _spec_rows, _spec_dp, _spec_b, _spec_w],
        out_specs=[_spec_rows, _spec_rows],
        out_shape=[jax.ShapeDtypeStruct((NPAD, D), _f32)] * 2,
    )(s, h, dp, b1, w2)


def _tc_post(s, h, dp, b2):
    return pl.pallas_call(
        _post_body, grid=_grid,
        in_specs=[_spec_s, _spec_rows, _spec_dp, _spec_b],
        out_specs=_spec_rows,
        out_shape=jax.ShapeDtypeStruct((NPAD, D), _f32),
    )(s, h, dp, b2)


def kernel(x, edge_index, W1, b1, W2, b2):
    src = edge_index[0].astype(jnp.int32)
    dst = edge_index[1].astype(jnp.int32)
    pad = jnp.full((EPAD - E,), N, dtype=jnp.int32)
    src_r = jnp.concatenate([src, pad]).reshape(CT, CHUNK)
    dst_r = jnp.concatenate([dst, pad]).reshape(CT, CHUNK)

    x_pad = jnp.pad(x, ((0, NPAD - N), (0, 0)))
    ones128 = jnp.ones((CHUNK, D), jnp.float32)
    zeros128 = jnp.zeros((RPT, D), jnp.float32)
    b1r = b1.reshape(1, D)
    b2r = b2.reshape(1, D)

    dp = _deg_kernel(dst_r, ones128, zeros128)
    h1, g1 = _tc_pre(x_pad, W1, dp)
    s1 = _agg_kernel(g1, src_r, dst_r, zeros128)
    h2, g2 = _tc_mid(s1, h1, dp, b1r, W2)
    s2 = _agg_kernel(g2, src_r, dst_r, zeros128)
    out = _tc_post(s2, h2, dp, b2r)
    return (out[:N], edge_index)
